# Initial kernel scaffold; baseline (speedup 1.0000x reference)
import jax
import jax.numpy as jnp
from jax.experimental import pallas as pl


def kernel(x, edge_index, edge_type_in, edge_attr, W_msg, rel_emb, W_rel, att_vec, bias):
    raise NotImplementedError("write your pallas kernel here")


# speedup vs baseline: 18.6547x; 18.6547x over previous
import functools

import jax
import jax.numpy as jnp
from jax import lax
from jax.experimental import pallas as pl
from jax.experimental.pallas import tpu as pltpu
from jax.experimental.pallas import tpu_sc as plsc

N = 10000
E = 320000
HID = 128
NUM_RELS = 16

NC = 2
NS = 16
NW = NC * NS

N_PAD = 10240
ROWS_PER_TILE = N_PAD // NS
E_PAD = 327680
EPW = E_PAD // NW
CHUNK = 128
NCHUNK = EPW // CHUNK


def _dense_body(x_ref, w_ref, att_ref, rel_ref, wrel_ref, conf_ref,
                h_ref, s_ref, c_ref, q_ref):
    x = x_ref[...]
    h = lax.dot_general(x, w_ref[...], (((1,), (1,)), ((), ())),
                        preferred_element_type=jnp.float32)
    h_ref[...] = h
    att = att_ref[...]
    s_ref[...] = lax.dot_general(h, att[0:2, :], (((1,), (1,)), ((), ())),
                                 preferred_element_type=jnp.float32)
    rproj = lax.dot_general(rel_ref[...], wrel_ref[...],
                            (((1,), (1,)), ((), ())),
                            preferred_element_type=jnp.float32)
    c_ref[...] = lax.dot_general(att[2:3, :], rproj, (((1,), (1,)), ((), ())),
                                 preferred_element_type=jnp.float32)
    q_ref[...] = jnp.sqrt(jnp.maximum(conf_ref[...], 1e-6))


_dense_call = pl.pallas_call(
    _dense_body,
    out_shape=[
        jax.ShapeDtypeStruct((N, HID), jnp.float32),
        jax.ShapeDtypeStruct((N, 2), jnp.float32),
        jax.ShapeDtypeStruct((1, NUM_RELS), jnp.float32),
        jax.ShapeDtypeStruct((E // HID, HID), jnp.float32),
    ],
)


_mesh = plsc.VectorSubcoreMesh(core_axis_name="c", subcore_axis_name="s")


@functools.partial(
    pl.kernel,
    out_type=[
        jax.ShapeDtypeStruct((NC, N_PAD, HID), jnp.float32),
        jax.ShapeDtypeStruct((NC, N_PAD), jnp.float32),
    ],
    mesh=_mesh,
    compiler_params=pltpu.CompilerParams(needs_layout_passes=False),
    scratch_types=[
        pltpu.VMEM_SHARED((N_PAD, HID), jnp.float32),
        pltpu.VMEM_SHARED((N_PAD,), jnp.float32),
        pltpu.VMEM((2 * N,), jnp.float32),
        pltpu.VMEM((NUM_RELS,), jnp.float32),
        pltpu.VMEM((CHUNK,), jnp.int32),
        pltpu.VMEM((CHUNK,), jnp.int32),
        pltpu.VMEM((CHUNK,), jnp.int32),
        pltpu.VMEM((CHUNK,), jnp.float32),
        pltpu.VMEM((CHUNK,), jnp.float32),
        pltpu.VMEM((CHUNK, HID), jnp.float32),
        pltpu.SemaphoreType.DMA,
    ],
)
def _sc_call(h_hbm, s_hbm, c_hbm, src_hbm, dst_hbm, t_hbm, q_hbm,
             acc_out, den_out,
             acc_sh, den_sh, s_v, c_v, src_v, dst_v, t_v, q_v, w_v, h_v,
             sem):
    cid = lax.axis_index("c")
    sid = lax.axis_index("s")

    zeros16 = jnp.zeros((16,), jnp.float32)

    def _zrow(j, carry):
        for k in range(HID // 16):
            h_v[j, pl.ds(k * 16, 16)] = zeros16
        return carry
    lax.fori_loop(0, CHUNK, _zrow, 0)
    for i in range(CHUNK // 16):
        w_v[pl.ds(i * 16, 16)] = zeros16
    for b in range(ROWS_PER_TILE // CHUNK):
        rs = sid * ROWS_PER_TILE + b * CHUNK
        pltpu.sync_copy(h_v, acc_sh.at[pl.ds(rs, CHUNK), :])
        pltpu.sync_copy(w_v, den_sh.at[pl.ds(rs, CHUNK)])

    pltpu.sync_copy(s_hbm, s_v)
    pltpu.sync_copy(c_hbm, c_v)

    plsc.subcore_barrier()

    wid = cid * NS + sid
    ebase = wid * EPW

    def _chunk(ch, carry):
        base = ebase + ch * CHUNK
        pltpu.sync_copy(src_hbm.at[pl.ds(base, CHUNK)], src_v)
        pltpu.sync_copy(dst_hbm.at[pl.ds(base, CHUNK)], dst_v)
        pltpu.sync_copy(t_hbm.at[pl.ds(base, CHUNK)], t_v)
        pltpu.sync_copy(q_hbm.at[pl.ds(base, CHUNK)], q_v)

        gcp = pltpu.async_copy(h_hbm.at[src_v], h_v, sem)

        for i in range(CHUNK // 16):
            sl = pl.ds(i * 16, 16)
            d16 = dst_v[sl]
            s16 = src_v[sl]
            t16 = jnp.clip(t_v[sl], 0, NUM_RELS - 1)
            sd = plsc.load_gather(s_v, [d16 * 2])
            ss = plsc.load_gather(s_v, [s16 * 2 + 1])
            cc = plsc.load_gather(c_v, [t16])
            e = sd + ss + cc
            e = jnp.maximum(e, 0.2 * e)
            w_v[sl] = jnp.exp(e) * q_v[sl]

        gcp.wait()

        def _sgroup(g, c2):
            wg = w_v[pl.ds(g * 16, 16)]
            for j in range(16):
                sw = wg[j]
                row = g * 16 + j
                for k in range(HID // 16):
                    slk = pl.ds(k * 16, 16)
                    h_v[row, slk] = h_v[row, slk] * sw
            return c2
        lax.fori_loop(0, CHUNK // 16, _sgroup, 0)

        pltpu.sync_copy(h_v, acc_sh.at[dst_v], add=True)
        pltpu.sync_copy(w_v, den_sh.at[dst_v], add=True)
        return carry

    lax.fori_loop(0, NCHUNK, _chunk, 0)

    plsc.subcore_barrier()

    for b in range(ROWS_PER_TILE // CHUNK):
        rs = sid * ROWS_PER_TILE + b * CHUNK
        pltpu.sync_copy(acc_sh.at[pl.ds(rs, CHUNK), :],
                        acc_out.at[cid, pl.ds(rs, CHUNK), :])
        pltpu.sync_copy(den_sh.at[pl.ds(rs, CHUNK)],
                        den_out.at[cid, pl.ds(rs, CHUNK)])


def _combine_body(acc_ref, den_ref, bias_ref, out_ref):
    a = acc_ref[0] + acc_ref[1]
    d = den_ref[0] + den_ref[1]
    out_ref[...] = a / (d + 1e-16) + bias_ref[...]


_combine_call = pl.pallas_call(
    _combine_body,
    out_shape=jax.ShapeDtypeStruct((N_PAD, HID), jnp.float32),
)


def kernel(x, edge_index, edge_type_in, edge_attr, W_msg, rel_emb, W_rel,
           att_vec, bias):
    att3 = att_vec.reshape(3, HID)
    conf2d = edge_attr.reshape(E // HID, HID)
    h, s, crel, q2d = _dense_call(x, W_msg, att3, rel_emb, W_rel, conf2d)

    pad = E_PAD - E
    spread = (jnp.arange(pad, dtype=jnp.int32) % N)
    src = jnp.concatenate([edge_index[0], spread])
    dst = jnp.concatenate([edge_index[1], spread])
    t = jnp.concatenate([edge_type_in, jnp.zeros((pad,), jnp.int32)])
    q = jnp.concatenate([q2d.reshape(E), jnp.zeros((pad,), jnp.float32)])

    acc, den = _sc_call(h, s.reshape(2 * N), crel.reshape(NUM_RELS),
                        src, dst, t, q)
    out_full = _combine_call(acc, den[:, :, None], bias.reshape(1, HID))
    return out_full[:N]


# speedup vs baseline: 40.5496x; 2.1737x over previous
import jax
import jax.numpy as jnp
from jax import lax
from jax.experimental import pallas as pl
from jax.experimental.pallas import tpu as pltpu
from jax.experimental.pallas import tpu_sc as plsc

N = 10000
E = 320000
HID = 128
NUM_RELS = 16

NC = 2
NS = 16
NW = NC * NS

N_PAD = 10240
ROWS_PER_TILE = N_PAD // NS
CHUNK = 128
NBUF = 2
NCHUNK = 80
EPW = NCHUNK * CHUNK
E_PAD = NW * EPW
KMAX = NCHUNK // NBUF


def _dense_body(x_ref, w_ref, att_ref, rel_ref, wrel_ref, conf_ref,
                h_ref, s_ref, c_ref, q_ref):
    x = x_ref[...]
    h = lax.dot_general(x, w_ref[...], (((1,), (1,)), ((), ())),
                        preferred_element_type=jnp.float32)
    h_ref[...] = h
    att = att_ref[...]
    s_ref[...] = lax.dot_general(att[0:2, :], h, (((1,), (1,)), ((), ())),
                                 preferred_element_type=jnp.float32)
    rproj = lax.dot_general(rel_ref[...], wrel_ref[...],
                            (((1,), (1,)), ((), ())),
                            preferred_element_type=jnp.float32)
    c_ref[...] = lax.dot_general(att[2:3, :], rproj, (((1,), (1,)), ((), ())),
                                 preferred_element_type=jnp.float32)
    q_ref[...] = jnp.sqrt(jnp.maximum(conf_ref[...], 1e-6))


_dense_call = pl.pallas_call(
    _dense_body,
    out_shape=[
        jax.ShapeDtypeStruct((N, HID), jnp.float32),
        jax.ShapeDtypeStruct((2, N), jnp.float32),
        jax.ShapeDtypeStruct((1, NUM_RELS), jnp.float32),
        jax.ShapeDtypeStruct((E // HID, HID), jnp.float32),
    ],
)


_mesh = plsc.VectorSubcoreMesh(core_axis_name="c", subcore_axis_name="s")

_sc_scratch = (
    [pltpu.VMEM_SHARED((N_PAD, HID), jnp.float32),
     pltpu.VMEM_SHARED((N_PAD,), jnp.float32),
     pltpu.VMEM_SHARED((N,), jnp.float32),
     pltpu.VMEM_SHARED((N,), jnp.float32),
     pltpu.VMEM((NUM_RELS,), jnp.float32)]
    + [pltpu.VMEM((CHUNK,), jnp.int32)] * NBUF
    + [pltpu.VMEM((CHUNK,), jnp.int32)] * NBUF
    + [pltpu.VMEM((CHUNK,), jnp.int32)] * NBUF
    + [pltpu.VMEM((CHUNK,), jnp.float32)] * NBUF
    + [pltpu.VMEM((CHUNK,), jnp.int32)] * NBUF
    + [pltpu.VMEM((CHUNK,), jnp.float32)] * NBUF
    + [pltpu.VMEM((CHUNK,), jnp.float32)] * NBUF
    + [pltpu.VMEM((CHUNK,), jnp.float32)] * NBUF
    + [pltpu.VMEM((CHUNK, HID), jnp.float32)] * NBUF
    + [pltpu.SemaphoreType.DMA] * (4 * NBUF)
)


def _sc_body(h_hbm, s_hbm, c_hbm, src_hbm, dst_hbm, t_hbm, q_hbm,
             acc_out, den_out, *scr):
    acc_sh, den_sh, s1_sh, s2_sh, c_v = scr[0:5]
    o = 5
    src_b = scr[o:o + NBUF]; o += NBUF
    dst_b = scr[o:o + NBUF]; o += NBUF
    t_b = scr[o:o + NBUF]; o += NBUF
    q_b = scr[o:o + NBUF]; o += NBUF
    ds_b = scr[o:o + NBUF]; o += NBUF
    w_b = scr[o:o + NBUF]; o += NBUF
    s1_b = scr[o:o + NBUF]; o += NBUF
    s2_b = scr[o:o + NBUF]; o += NBUF
    h_b = scr[o:o + NBUF]; o += NBUF
    semi = scr[o:o + NBUF]; o += NBUF
    semg = scr[o:o + NBUF]; o += NBUF
    semt = scr[o:o + NBUF]; o += NBUF
    sems = scr[o:o + NBUF]; o += NBUF

    cid = lax.axis_index("c")
    sid = lax.axis_index("s")
    zeros16 = jnp.zeros((16,), jnp.float32)

    wid = cid * NS + sid
    ebase = wid * EPW

    def idx_issue(p, ch):
        base = ebase + ch * CHUNK
        pltpu.async_copy(src_hbm.at[pl.ds(base, CHUNK)], src_b[p], semi[p])
        pltpu.async_copy(dst_hbm.at[pl.ds(base, CHUNK)], dst_b[p], semi[p])
        pltpu.async_copy(t_hbm.at[pl.ds(base, CHUNK)], t_b[p], semi[p])
        pltpu.async_copy(q_hbm.at[pl.ds(base, CHUNK)], q_b[p], semi[p])

    def idx_drain(p):
        pltpu.make_async_copy(src_hbm.at[pl.ds(0, CHUNK)], src_b[p], semi[p]).wait()
        pltpu.make_async_copy(dst_hbm.at[pl.ds(0, CHUNK)], dst_b[p], semi[p]).wait()
        pltpu.make_async_copy(t_hbm.at[pl.ds(0, CHUNK)], t_b[p], semi[p]).wait()
        pltpu.make_async_copy(q_hbm.at[pl.ds(0, CHUNK)], q_b[p], semi[p]).wait()

    def gather_issue(p):
        pltpu.async_copy(h_hbm.at[src_b[p]], h_b[p], semg[p])

    def gather_wait(p):
        pltpu.make_async_copy(h_hbm.at[src_b[p]], h_b[p], semg[p]).wait()

    def sgather_issue(p):
        pltpu.async_copy(s1_sh.at[dst_b[p]], s1_b[p], semt[p])
        pltpu.async_copy(s2_sh.at[src_b[p]], s2_b[p], semt[p])

    def sgather_drain(p):
        pltpu.make_async_copy(s1_sh.at[dst_b[p]], s1_b[p], semt[p]).wait()
        pltpu.make_async_copy(s2_sh.at[src_b[p]], s2_b[p], semt[p]).wait()

    def scat_issue(p):
        pltpu.async_copy(h_b[p], acc_sh.at[ds_b[p]], sems[p], add=True)
        pltpu.async_copy(w_b[p], den_sh.at[ds_b[p]], sems[p], add=True)

    def scat_drain(p):
        pltpu.make_async_copy(h_b[p], acc_sh.at[ds_b[p]], sems[p]).wait()
        pltpu.make_async_copy(w_b[p], den_sh.at[ds_b[p]], sems[p]).wait()

    def compute_w(p):
        for i in range(CHUNK // 16):
            sl = pl.ds(i * 16, 16)
            t16 = jnp.clip(t_b[p][sl], 0, NUM_RELS - 1)
            cc = plsc.load_gather(c_v, [t16])
            e = s1_b[p][sl] + s2_b[p][sl] + cc
            e = jnp.maximum(e, 0.2 * e)
            w_b[p][sl] = jnp.exp(e) * q_b[p][sl]
            ds_b[p][sl] = dst_b[p][sl]

    def scale(p):
        def _sgroup(g, c2):
            wg = w_b[p][pl.ds(g * 16, 16)]
            for j in range(16):
                sw = wg[j]
                row = g * 16 + j
                for k in range(HID // 16):
                    slk = pl.ds(k * 16, 16)
                    h_b[p][row, slk] = h_b[p][row, slk] * sw
            return c2
        lax.fori_loop(0, CHUNK // 16, _sgroup, 0)

    for p in range(NBUF):
        idx_issue(p, p)

    def _zrow(j, carry):
        for k in range(HID // 16):
            h_b[0][j, pl.ds(k * 16, 16)] = zeros16
        return carry
    lax.fori_loop(0, CHUNK, _zrow, 0)
    for i in range(CHUNK // 16):
        w_b[0][pl.ds(i * 16, 16)] = zeros16
    for b in range(ROWS_PER_TILE // CHUNK):
        rs = sid * ROWS_PER_TILE + b * CHUNK
        pltpu.sync_copy(h_b[0], acc_sh.at[pl.ds(rs, CHUNK), :])
        pltpu.sync_copy(w_b[0], den_sh.at[pl.ds(rs, CHUNK)])

    @pl.when(sid == 0)
    def _():
        pltpu.sync_copy(s_hbm.at[0], s1_sh)
        pltpu.sync_copy(s_hbm.at[1], s2_sh)
    pltpu.sync_copy(c_hbm, c_v)

    idx_drain(0)
    gather_issue(0)
    plsc.subcore_barrier()
    sgather_issue(0)

    def body(k, carry):
        for j in range(NBUF):
            ch = k * NBUF + j
            p = j
            p1 = (j + 1) % NBUF
            sgather_drain(p)
            compute_w(p)
            if j == NBUF - 1:
                scat_drain(p1)

                @pl.when(k < KMAX - 1)
                def _():
                    idx_drain(p1)
                    gather_issue(p1)
                    sgather_issue(p1)
            else:
                @pl.when(k > 0)
                def _():
                    scat_drain(p1)
                idx_drain(p1)
                gather_issue(p1)
                sgather_issue(p1)
            gather_wait(p)
            scale(p)
            scat_issue(p)

            @pl.when(k < KMAX - 1)
            def _():
                idx_issue(p, ch + NBUF)
        return carry

    lax.fori_loop(0, KMAX, body, 0)

    scat_drain(1)

    plsc.subcore_barrier()

    for b in range(ROWS_PER_TILE // CHUNK):
        rs = sid * ROWS_PER_TILE + b * CHUNK
        pltpu.sync_copy(acc_sh.at[pl.ds(rs, CHUNK), :],
                        acc_out.at[cid, pl.ds(rs, CHUNK), :])
        pltpu.sync_copy(den_sh.at[pl.ds(rs, CHUNK)],
                        den_out.at[cid, pl.ds(rs, CHUNK)])


_sc_call = pl.kernel(
    _sc_body,
    out_type=[
        jax.ShapeDtypeStruct((NC, N_PAD, HID), jnp.float32),
        jax.ShapeDtypeStruct((NC, N_PAD), jnp.float32),
    ],
    mesh=_mesh,
    compiler_params=pltpu.CompilerParams(needs_layout_passes=False),
    scratch_types=_sc_scratch,
)


def _combine_body(acc_ref, den_ref, bias_ref, out_ref):
    a = acc_ref[0] + acc_ref[1]
    d = den_ref[0] + den_ref[1]
    out_ref[...] = a / (d + 1e-16) + bias_ref[...]


_combine_call = pl.pallas_call(
    _combine_body,
    out_shape=jax.ShapeDtypeStruct((N_PAD, HID), jnp.float32),
)


def kernel(x, edge_index, edge_type_in, edge_attr, W_msg, rel_emb, W_rel,
           att_vec, bias):
    att3 = att_vec.reshape(3, HID)
    conf2d = edge_attr.reshape(E // HID, HID)
    h, s, crel, q2d = _dense_call(x, W_msg, att3, rel_emb, W_rel, conf2d)

    pad = E_PAD - E
    spread = (jnp.arange(pad, dtype=jnp.int32) % N)
    src = jnp.concatenate([edge_index[0], spread])
    dst = jnp.concatenate([edge_index[1], spread])
    t = jnp.concatenate([edge_type_in, jnp.zeros((pad,), jnp.int32)])
    q = jnp.concatenate([q2d.reshape(E), jnp.zeros((pad,), jnp.float32)])

    acc, den = _sc_call(h, s, crel.reshape(NUM_RELS), src, dst, t, q)
    out_full = _combine_call(acc, den[:, :, None], bias.reshape(1, HID))
    return out_full[:N]


# speedup vs baseline: 43.8746x; 1.0820x over previous
import jax
import jax.numpy as jnp
from jax import lax
from jax.experimental import pallas as pl
from jax.experimental.pallas import tpu as pltpu
from jax.experimental.pallas import tpu_sc as plsc

N = 10000
E = 320000
HID = 128
NUM_RELS = 16

NC = 2
NS = 16
NW = NC * NS

N_PAD = 10240
ROWS_PER_TILE = N_PAD // NS
CHUNK = 128
NBUF = 2
NCHUNK = 80
EPW = NCHUNK * CHUNK
E_PAD = NW * EPW
KMAX = NCHUNK // NBUF


_EROWS = E // HID
_EROWS_PAD = E_PAD // HID


def _dense_body(x_ref, w_ref, att_ref, rel_ref, wrel_ref, conf_ref,
                ei_ref, et_ref,
                h_ref, s_ref, c_ref, q_ref, src_ref, dst_ref, t_ref):
    x = x_ref[...]
    h = lax.dot_general(x, w_ref[...], (((1,), (1,)), ((), ())),
                        preferred_element_type=jnp.float32)
    h_ref[...] = h
    att = att_ref[...]
    s_ref[...] = lax.dot_general(att[0:2, :], h, (((1,), (1,)), ((), ())),
                                 preferred_element_type=jnp.float32)
    rproj = lax.dot_general(rel_ref[...], wrel_ref[...],
                            (((1,), (1,)), ((), ())),
                            preferred_element_type=jnp.float32)
    c_ref[...] = lax.dot_general(att[2:3, :], rproj, (((1,), (1,)), ((), ())),
                                 preferred_element_type=jnp.float32)
    q_ref[0:_EROWS, :] = jnp.sqrt(jnp.maximum(conf_ref[...], 1e-6))
    q_ref[_EROWS:_EROWS_PAD, :] = jnp.zeros(
        (_EROWS_PAD - _EROWS, HID), jnp.float32)
    npad_rows = _EROWS_PAD - _EROWS
    spread = (lax.broadcasted_iota(jnp.int32, (npad_rows, HID), 0) * HID
              + lax.broadcasted_iota(jnp.int32, (npad_rows, HID), 1)) % N
    src_ref[0:_EROWS, :] = ei_ref[0]
    src_ref[_EROWS:_EROWS_PAD, :] = spread
    dst_ref[0:_EROWS, :] = ei_ref[1]
    dst_ref[_EROWS:_EROWS_PAD, :] = spread
    t_ref[0:_EROWS, :] = et_ref[...]
    t_ref[_EROWS:_EROWS_PAD, :] = jnp.zeros((npad_rows, HID), jnp.int32)


_dense_call = pl.pallas_call(
    _dense_body,
    out_shape=[
        jax.ShapeDtypeStruct((N, HID), jnp.float32),
        jax.ShapeDtypeStruct((2, N), jnp.float32),
        jax.ShapeDtypeStruct((1, NUM_RELS), jnp.float32),
        jax.ShapeDtypeStruct((_EROWS_PAD, HID), jnp.float32),
        jax.ShapeDtypeStruct((_EROWS_PAD, HID), jnp.int32),
        jax.ShapeDtypeStruct((_EROWS_PAD, HID), jnp.int32),
        jax.ShapeDtypeStruct((_EROWS_PAD, HID), jnp.int32),
    ],
)


_mesh = plsc.VectorSubcoreMesh(core_axis_name="c", subcore_axis_name="s")

_sc_scratch = (
    [pltpu.VMEM_SHARED((N_PAD, HID), jnp.float32),
     pltpu.VMEM_SHARED((N_PAD,), jnp.float32),
     pltpu.VMEM_SHARED((N,), jnp.float32),
     pltpu.VMEM_SHARED((N,), jnp.float32),
     pltpu.VMEM((NUM_RELS,), jnp.float32)]
    + [pltpu.VMEM((CHUNK,), jnp.int32)] * NBUF
    + [pltpu.VMEM((CHUNK,), jnp.int32)] * NBUF
    + [pltpu.VMEM((CHUNK,), jnp.int32)] * NBUF
    + [pltpu.VMEM((CHUNK,), jnp.float32)] * NBUF
    + [pltpu.VMEM((CHUNK,), jnp.int32)] * NBUF
    + [pltpu.VMEM((CHUNK,), jnp.float32)] * NBUF
    + [pltpu.VMEM((CHUNK,), jnp.float32)] * NBUF
    + [pltpu.VMEM((CHUNK,), jnp.float32)] * NBUF
    + [pltpu.VMEM((CHUNK, HID), jnp.float32)] * NBUF
    + [pltpu.SemaphoreType.DMA] * (4 * NBUF)
)


def _sc_body(h_hbm, s_hbm, c_hbm, src_hbm, dst_hbm, t_hbm, q_hbm,
             acc_out, den_out, *scr):
    acc_sh, den_sh, s1_sh, s2_sh, c_v = scr[0:5]
    o = 5
    src_b = scr[o:o + NBUF]; o += NBUF
    dst_b = scr[o:o + NBUF]; o += NBUF
    t_b = scr[o:o + NBUF]; o += NBUF
    q_b = scr[o:o + NBUF]; o += NBUF
    ds_b = scr[o:o + NBUF]; o += NBUF
    w_b = scr[o:o + NBUF]; o += NBUF
    s1_b = scr[o:o + NBUF]; o += NBUF
    s2_b = scr[o:o + NBUF]; o += NBUF
    h_b = scr[o:o + NBUF]; o += NBUF
    semi = scr[o:o + NBUF]; o += NBUF
    semg = scr[o:o + NBUF]; o += NBUF
    semt = scr[o:o + NBUF]; o += NBUF
    sems = scr[o:o + NBUF]; o += NBUF

    cid = lax.axis_index("c")
    sid = lax.axis_index("s")
    zeros16 = jnp.zeros((16,), jnp.float32)

    wid = cid * NS + sid
    ebase = wid * EPW

    def idx_issue(p, ch):
        base = ebase + ch * CHUNK
        pltpu.async_copy(src_hbm.at[pl.ds(base, CHUNK)], src_b[p], semi[p])
        pltpu.async_copy(dst_hbm.at[pl.ds(base, CHUNK)], dst_b[p], semi[p])
        pltpu.async_copy(t_hbm.at[pl.ds(base, CHUNK)], t_b[p], semi[p])
        pltpu.async_copy(q_hbm.at[pl.ds(base, CHUNK)], q_b[p], semi[p])

    def idx_drain(p):
        pltpu.make_async_copy(src_hbm.at[pl.ds(0, CHUNK)], src_b[p], semi[p]).wait()
        pltpu.make_async_copy(dst_hbm.at[pl.ds(0, CHUNK)], dst_b[p], semi[p]).wait()
        pltpu.make_async_copy(t_hbm.at[pl.ds(0, CHUNK)], t_b[p], semi[p]).wait()
        pltpu.make_async_copy(q_hbm.at[pl.ds(0, CHUNK)], q_b[p], semi[p]).wait()

    def gather_issue(p):
        pltpu.async_copy(h_hbm.at[src_b[p]], h_b[p], semg[p])

    def gather_wait(p):
        pltpu.make_async_copy(h_hbm.at[src_b[p]], h_b[p], semg[p]).wait()

    def sgather_issue(p):
        pltpu.async_copy(s1_sh.at[dst_b[p]], s1_b[p], semt[p])
        pltpu.async_copy(s2_sh.at[src_b[p]], s2_b[p], semt[p])

    def sgather_drain(p):
        pltpu.make_async_copy(s1_sh.at[dst_b[p]], s1_b[p], semt[p]).wait()
        pltpu.make_async_copy(s2_sh.at[src_b[p]], s2_b[p], semt[p]).wait()

    def scat_issue(p):
        pltpu.async_copy(h_b[p], acc_sh.at[ds_b[p]], sems[p], add=True)
        pltpu.async_copy(w_b[p], den_sh.at[ds_b[p]], sems[p], add=True)

    def scat_drain(p):
        pltpu.make_async_copy(h_b[p], acc_sh.at[ds_b[p]], sems[p]).wait()
        pltpu.make_async_copy(w_b[p], den_sh.at[ds_b[p]], sems[p]).wait()

    def compute_w(p):
        for i in range(CHUNK // 16):
            sl = pl.ds(i * 16, 16)
            t16 = jnp.clip(t_b[p][sl], 0, NUM_RELS - 1)
            cc = plsc.load_gather(c_v, [t16])
            e = s1_b[p][sl] + s2_b[p][sl] + cc
            e = jnp.maximum(e, 0.2 * e)
            w_b[p][sl] = jnp.exp(e) * q_b[p][sl]
            ds_b[p][sl] = dst_b[p][sl]

    def scale(p):
        def _sgroup(g, c2):
            wg = w_b[p][pl.ds(g * 16, 16)]
            for j in range(16):
                sw = wg[j]
                row = g * 16 + j
                for k in range(HID // 16):
                    slk = pl.ds(k * 16, 16)
                    h_b[p][row, slk] = h_b[p][row, slk] * sw
            return c2
        lax.fori_loop(0, CHUNK // 16, _sgroup, 0)

    for p in range(NBUF):
        idx_issue(p, p)

    def _zrow(j, carry):
        for k in range(HID // 16):
            h_b[0][j, pl.ds(k * 16, 16)] = zeros16
        return carry
    lax.fori_loop(0, CHUNK, _zrow, 0)
    for i in range(CHUNK // 16):
        w_b[0][pl.ds(i * 16, 16)] = zeros16
    for b in range(ROWS_PER_TILE // CHUNK):
        rs = sid * ROWS_PER_TILE + b * CHUNK
        pltpu.sync_copy(h_b[0], acc_sh.at[pl.ds(rs, CHUNK), :])
        pltpu.sync_copy(w_b[0], den_sh.at[pl.ds(rs, CHUNK)])

    @pl.when(sid == 0)
    def _():
        pltpu.sync_copy(s_hbm.at[0], s1_sh)
        pltpu.sync_copy(s_hbm.at[1], s2_sh)
    pltpu.sync_copy(c_hbm, c_v)

    idx_drain(0)
    gather_issue(0)
    plsc.subcore_barrier()
    sgather_issue(0)

    def body(k, carry):
        for j in range(NBUF):
            ch = k * NBUF + j
            p = j
            p1 = (j + 1) % NBUF
            sgather_drain(p)
            compute_w(p)
            if j == NBUF - 1:
                scat_drain(p1)

                @pl.when(k < KMAX - 1)
                def _():
                    idx_drain(p1)
                    gather_issue(p1)
                    sgather_issue(p1)
            else:
                @pl.when(k > 0)
                def _():
                    scat_drain(p1)
                idx_drain(p1)
                gather_issue(p1)
                sgather_issue(p1)
            gather_wait(p)
            scale(p)
            scat_issue(p)

            @pl.when(k < KMAX - 1)
            def _():
                idx_issue(p, ch + NBUF)
        return carry

    lax.fori_loop(0, KMAX, body, 0)

    scat_drain(1)

    plsc.subcore_barrier()

    for b in range(ROWS_PER_TILE // CHUNK):
        rs = sid * ROWS_PER_TILE + b * CHUNK
        pltpu.sync_copy(acc_sh.at[pl.ds(rs, CHUNK), :],
                        acc_out.at[cid, pl.ds(rs, CHUNK), :])
        pltpu.sync_copy(den_sh.at[pl.ds(rs, CHUNK)],
                        den_out.at[cid, pl.ds(rs, CHUNK)])


_sc_call = pl.kernel(
    _sc_body,
    out_type=[
        jax.ShapeDtypeStruct((NC, N_PAD, HID), jnp.float32),
        jax.ShapeDtypeStruct((NC, N_PAD), jnp.float32),
    ],
    mesh=_mesh,
    compiler_params=pltpu.CompilerParams(needs_layout_passes=False),
    scratch_types=_sc_scratch,
)


def _combine_body(acc_ref, den_ref, bias_ref, out_ref):
    a = acc_ref[0, 0:N, :] + acc_ref[1, 0:N, :]
    d = den_ref[0, 0:N, :] + den_ref[1, 0:N, :]
    out_ref[...] = a / (d + 1e-16) + bias_ref[...]


_combine_call = pl.pallas_call(
    _combine_body,
    out_shape=jax.ShapeDtypeStruct((N, HID), jnp.float32),
)


def kernel(x, edge_index, edge_type_in, edge_attr, W_msg, rel_emb, W_rel,
           att_vec, bias):
    att3 = att_vec.reshape(3, HID)
    conf2d = edge_attr.reshape(E // HID, HID)
    ei2d = edge_index.reshape(2, _EROWS, HID)
    et2d = edge_type_in.reshape(_EROWS, HID)
    h, s, crel, q2d, src2d, dst2d, t2d = _dense_call(
        x, W_msg, att3, rel_emb, W_rel, conf2d, ei2d, et2d)

    acc, den = _sc_call(h, s, crel.reshape(NUM_RELS),
                        src2d.reshape(E_PAD), dst2d.reshape(E_PAD),
                        t2d.reshape(E_PAD), q2d.reshape(E_PAD))
    return _combine_call(acc, den[:, :, None], bias.reshape(1, HID))
